# baseline (speedup 1.0000x reference)
import jax
import jax.numpy as jnp
from jax import lax
from jax.experimental import pallas as pl
from jax.experimental.pallas import tpu as pltpu
from jax.experimental.pallas import tpu_sc as plsc

T = 2048
D = 1024
F = 1408
E = 8
TOPK = 2

BLK = 256
P = T * TOPK + E * BLK
NB = P // BLK

LANES = 128
NEG = -1e30

NC = 2
NS = 16
NW = NC * NS
SCL = 16

CHUNK = P // NW
GR = 64
NGB = 3
TPS = T // NS
TCH = T // NW
CCH = 8


def _router_body(x_ref, gate_ref, pos1_ref, pos2_ref, wn1_ref, wn2_ref,
                 ends_ref):
    x = x_ref[...]
    logits = lax.dot_general(
        x, gate_ref[...], (((1,), (1,)), ((), ())),
        preferred_element_type=jnp.float32)
    logits = jnp.pad(logits, ((0, 0), (0, LANES - E)))
    eio = lax.broadcasted_iota(jnp.int32, (T, LANES), 1)
    valid = eio < E
    l = jnp.where(valid, logits, NEG)
    m1 = jnp.max(l, axis=1, keepdims=True)
    idx1 = jnp.min(jnp.where((l == m1) & valid, eio, LANES - 1), axis=1,
                   keepdims=True)
    sel1 = eio == idx1
    l2 = jnp.where(sel1, NEG, l)
    m2 = jnp.max(l2, axis=1, keepdims=True)
    idx2 = jnp.min(jnp.where((l2 == m2) & valid, eio, LANES - 1), axis=1,
                   keepdims=True)
    sel2 = eio == idx2
    wn1_ref[...] = jax.nn.sigmoid(m1 - m2)
    wn2_ref[...] = jax.nn.sigmoid(m2 - m1)

    a1 = sel1.astype(jnp.float32)
    a2 = sel2.astype(jnp.float32)
    s = a1 + a2
    ri = lax.broadcasted_iota(jnp.int32, (512, 512), 0)
    ci = lax.broadcasted_iota(jnp.int32, (512, 512), 1)
    tri = (ri > ci).astype(jnp.float32)
    off = jnp.zeros((1, LANES), jnp.float32)
    cs = []
    for g in range(4):
        sg = s[g * 512:(g + 1) * 512, :]
        cs.append(jnp.dot(tri, sg, preferred_element_type=jnp.float32) + off)
        off = off + jnp.sum(sg, axis=0, keepdims=True)
    c = jnp.concatenate(cs, axis=0)
    counts = off
    pc = jnp.ceil(counts * (1.0 / BLK)) * BLK
    rl = lax.broadcasted_iota(jnp.int32, (LANES, LANES), 0)
    cl = lax.broadcasted_iota(jnp.int32, (LANES, LANES), 1)
    tril = (rl < cl).astype(jnp.float32)
    starts = jnp.dot(pc, tril, preferred_element_type=jnp.float32)
    ends_ref[...] = (starts + pc).astype(jnp.int32)

    rank1 = jnp.sum(a1 * c, axis=1, keepdims=True)
    rank2 = jnp.sum(a2 * c, axis=1, keepdims=True)
    st1 = jnp.sum(a1 * starts, axis=1, keepdims=True)
    st2 = jnp.sum(a2 * starts, axis=1, keepdims=True)
    pos1_ref[...] = (st1 + rank1).astype(jnp.int32)
    pos2_ref[...] = (st2 + rank2).astype(jnp.int32)


def _router(x, gate_w):
    return pl.pallas_call(
        _router_body,
        out_shape=(
            jax.ShapeDtypeStruct((T, 1), jnp.int32),
            jax.ShapeDtypeStruct((T, 1), jnp.int32),
            jax.ShapeDtypeStruct((T, 1), jnp.float32),
            jax.ShapeDtypeStruct((T, 1), jnp.float32),
            jax.ShapeDtypeStruct((1, LANES), jnp.int32),
        ),
    )(x, gate_w)


SCH = 16
NCH = TCH // SCH


def _dispatch_body(x_hbm, pos1_hbm, pos2_hbm, wn1_hbm, wn2_hbm, xs_hbm, ws_hbm,
                   p1_v, p2_v, w1_v, w2_v, rows, isems, o1sems, o2sems, wsems):
    cid = lax.axis_index("c")
    sid = lax.axis_index("s")
    wid = sid * NC + cid
    tb = wid * TCH

    pltpu.sync_copy(pos1_hbm.at[pl.ds(tb, TCH)], p1_v)
    pltpu.sync_copy(pos2_hbm.at[pl.ds(tb, TCH)], p2_v)
    pltpu.sync_copy(wn1_hbm.at[pl.ds(tb, TCH)], w1_v)
    pltpu.sync_copy(wn2_hbm.at[pl.ds(tb, TCH)], w2_v)

    wc1 = pltpu.async_copy(w1_v, ws_hbm.at[p1_v], wsems[0])
    wc2 = pltpu.async_copy(w2_v, ws_hbm.at[p2_v], wsems[1])

    icp = {}
    s1 = {}
    s2 = {}

    def start_in(ch):
        b = ch % 2
        icp[ch] = pltpu.async_copy(
            x_hbm.at[pl.ds(tb + ch * SCH, SCH)], rows[b], isems[b])

    start_in(0)
    for ch in range(NCH):
        if ch + 1 < NCH:
            if ch + 1 >= 2:
                s1[ch - 1].wait()
                s2[ch - 1].wait()
            start_in(ch + 1)
        b = ch % 2
        icp[ch].wait()
        idx1 = p1_v[pl.ds(ch * SCH, SCH)]
        idx2 = p2_v[pl.ds(ch * SCH, SCH)]
        s1[ch] = pltpu.async_copy(rows[b], xs_hbm.at[idx1], o1sems[b])
        s2[ch] = pltpu.async_copy(rows[b], xs_hbm.at[idx2], o2sems[b])
    for ch in (NCH - 2, NCH - 1):
        s1[ch].wait()
        s2[ch].wait()
    wc1.wait()
    wc2.wait()


def _dispatch(x, pos1, pos2, wn1, wn2):
    return pl.kernel(
        _dispatch_body,
        out_type=(
            jax.ShapeDtypeStruct((P, D), jnp.float32),
            jax.ShapeDtypeStruct((P,), jnp.float32),
        ),
        mesh=plsc.VectorSubcoreMesh(core_axis_name="c", subcore_axis_name="s"),
        compiler_params=pltpu.CompilerParams(needs_layout_passes=False),
        scratch_types=[
            pltpu.VMEM((TCH,), jnp.int32),
            pltpu.VMEM((TCH,), jnp.int32),
            pltpu.VMEM((TCH,), jnp.float32),
            pltpu.VMEM((TCH,), jnp.float32),
            [pltpu.VMEM((SCH, D), jnp.float32) for _ in range(2)],
            [pltpu.SemaphoreType.DMA for _ in range(2)],
            [pltpu.SemaphoreType.DMA for _ in range(2)],
            [pltpu.SemaphoreType.DMA for _ in range(2)],
            [pltpu.SemaphoreType.DMA for _ in range(2)],
        ],
    )(x, pos1, pos2, wn1, wn2)


def _dot_bf16(a, b):
    return lax.dot_general(
        a, b, (((1,), (0,)), ((), ())),
        precision=lax.Precision.DEFAULT,
        preferred_element_type=jnp.float32)


def _mlp_body(bm_ref, xs_ref, ws_ref, w1_ref, w2_ref, ys_ref):
    @pl.when(pl.program_id(0) < bm_ref[NB])
    def _():
        h = _dot_bf16(xs_ref[...], w1_ref[0])
        act = h * jax.nn.sigmoid(h)
        act = act * ws_ref[...]
        ys_ref[...] = _dot_bf16(act, w2_ref[0])


def _mlp(bm, xs, ws, w1, w2):
    def _rowmap(b, bm):
        return (jnp.minimum(b, bm[NB] - 1), 0)

    grid_spec = pltpu.PrefetchScalarGridSpec(
        num_scalar_prefetch=1,
        grid=(NB,),
        in_specs=[
            pl.BlockSpec((BLK, D), _rowmap),
            pl.BlockSpec((BLK, 1), _rowmap),
            pl.BlockSpec((1, D, F), lambda b, bm: (bm[b], 0, 0)),
            pl.BlockSpec((1, F, D), lambda b, bm: (bm[b], 0, 0)),
        ],
        out_specs=pl.BlockSpec((BLK, D), _rowmap),
    )
    return pl.pallas_call(
        _mlp_body,
        grid_spec=grid_spec,
        out_shape=jax.ShapeDtypeStruct((P, D), jnp.float32),
        compiler_params=pltpu.CompilerParams(
            dimension_semantics=("arbitrary",),
        ),
    )(bm, xs, ws, w1, w2)


NCB = 6


def _combine_body(ys_hbm, pos1_hbm, pos2_hbm, out_hbm,
                  i1, i2, r1, r2, g1sems, g2sems, outsems):
    wid = lax.axis_index("s") * NC + lax.axis_index("c")
    base = wid * TCH
    nch = TCH // CCH
    pltpu.sync_copy(pos1_hbm.at[pl.ds(base, TCH)], i1)
    pltpu.sync_copy(pos2_hbm.at[pl.ds(base, TCH)], i2)

    g1 = {}
    g2 = {}
    oc = {}

    def fire(ch):
        b = ch % NCB
        g1[ch] = pltpu.async_copy(
            ys_hbm.at[i1.at[pl.ds(ch * CCH, CCH)]], r1[b], g1sems[b])
        g2[ch] = pltpu.async_copy(
            ys_hbm.at[i2.at[pl.ds(ch * CCH, CCH)]], r2[b], g2sems[b])

    for ch in range(min(NCB, nch)):
        fire(ch)
    for ch in range(nch):
        b = ch % NCB
        if ch >= 2:
            oc[ch - 2].wait()
            if ch + NCB - 2 < nch:
                fire(ch + NCB - 2)
        g1[ch].wait()
        g2[ch].wait()
        for r in range(CCH):
            def addvec(j, carry, r=r):
                for u in range(4):
                    sl = pl.ds((j * 4 + u) * SCL, SCL)
                    r1[b][r, sl] = r1[b][r, sl] + r2[b][r, sl]
                return carry
            lax.fori_loop(0, D // (SCL * 4), addvec, 0)
        oc[ch] = pltpu.async_copy(
            r1[b], out_hbm.at[pl.ds(base + ch * CCH, CCH)], outsems[ch % 2])
    for ch in (nch - 2, nch - 1):
        oc[ch].wait()


def _combine(ys, pos1, pos2):
    return pl.kernel(
        _combine_body,
        out_type=jax.ShapeDtypeStruct((T, D), jnp.float32),
        mesh=plsc.VectorSubcoreMesh(core_axis_name="c", subcore_axis_name="s"),
        compiler_params=pltpu.CompilerParams(needs_layout_passes=False),
        scratch_types=[
            pltpu.VMEM((TCH,), jnp.int32),
            pltpu.VMEM((TCH,), jnp.int32),
            [pltpu.VMEM((CCH, D), jnp.float32) for _ in range(NCB)],
            [pltpu.VMEM((CCH, D), jnp.float32) for _ in range(NCB)],
            [pltpu.SemaphoreType.DMA for _ in range(NCB)],
            [pltpu.SemaphoreType.DMA for _ in range(NCB)],
            [pltpu.SemaphoreType.DMA for _ in range(2)],
        ],
    )(ys, pos1, pos2)


def kernel(hidden_states, gate_w, experts_w1, experts_w2):
    pos1, pos2, wn1, wn2, ends = _router(hidden_states, gate_w)
    pos1f = pos1.reshape(T)
    pos2f = pos2.reshape(T)

    ends8 = ends[0, :E]
    bidx = jnp.arange(NB, dtype=jnp.int32) * BLK
    be = jnp.clip(jnp.sum((ends8[None, :] <= bidx[:, None]).astype(jnp.int32),
                          axis=1), 0, E - 1).astype(jnp.int32)
    nact = ends8[E - 1] // BLK
    be_sk = be[jnp.minimum(jnp.arange(NB), nact - 1)]
    bm = jnp.concatenate([be_sk, nact[None]]).astype(jnp.int32)

    xs, ws = _dispatch(hidden_states, pos1f, pos2f,
                       wn1.reshape(T), wn2.reshape(T))
    ys = _mlp(bm, xs, ws.reshape(P, 1), experts_w1, experts_w2)
    return _combine(ys, pos1f, pos2f)


# speedup vs baseline: 1.1484x; 1.1484x over previous
import jax
import jax.numpy as jnp
from jax import lax
from jax.experimental import pallas as pl
from jax.experimental.pallas import tpu as pltpu
from jax.experimental.pallas import tpu_sc as plsc

T = 2048
D = 1024
F = 1408
E = 8
TOPK = 2

BLK = 256
P = T * TOPK + E * BLK
NB = P // BLK

LANES = 128
NEG = -1e30

NC = 2
NS = 16
NW = NC * NS
SCL = 16

CHUNK = P // NW
GR = 64
NGB = 3
TPS = T // NS
TCH = T // NW
CCH = 8


def _router_body(x_ref, gate_ref, pos1_ref, pos2_ref, wn1_ref, wn2_ref,
                 ends_ref):
    x = x_ref[...]
    logits = lax.dot_general(
        x, gate_ref[...], (((1,), (1,)), ((), ())),
        preferred_element_type=jnp.float32)
    logits = jnp.pad(logits, ((0, 0), (0, LANES - E)))
    eio = lax.broadcasted_iota(jnp.int32, (T, LANES), 1)
    valid = eio < E
    l = jnp.where(valid, logits, NEG)
    m1 = jnp.max(l, axis=1, keepdims=True)
    idx1 = jnp.min(jnp.where((l == m1) & valid, eio, LANES - 1), axis=1,
                   keepdims=True)
    sel1 = eio == idx1
    l2 = jnp.where(sel1, NEG, l)
    m2 = jnp.max(l2, axis=1, keepdims=True)
    idx2 = jnp.min(jnp.where((l2 == m2) & valid, eio, LANES - 1), axis=1,
                   keepdims=True)
    sel2 = eio == idx2
    wn1_ref[...] = jax.nn.sigmoid(m1 - m2)
    wn2_ref[...] = jax.nn.sigmoid(m2 - m1)

    a1 = sel1.astype(jnp.float32)
    a2 = sel2.astype(jnp.float32)
    s = a1 + a2
    ri = lax.broadcasted_iota(jnp.int32, (512, 512), 0)
    ci = lax.broadcasted_iota(jnp.int32, (512, 512), 1)
    tri = (ri > ci).astype(jnp.float32)
    off = jnp.zeros((1, LANES), jnp.float32)
    cs = []
    for g in range(4):
        sg = s[g * 512:(g + 1) * 512, :]
        cs.append(jnp.dot(tri, sg, preferred_element_type=jnp.float32) + off)
        off = off + jnp.sum(sg, axis=0, keepdims=True)
    c = jnp.concatenate(cs, axis=0)
    counts = off
    pc = jnp.ceil(counts * (1.0 / BLK)) * BLK
    rl = lax.broadcasted_iota(jnp.int32, (LANES, LANES), 0)
    cl = lax.broadcasted_iota(jnp.int32, (LANES, LANES), 1)
    tril = (rl < cl).astype(jnp.float32)
    starts = jnp.dot(pc, tril, preferred_element_type=jnp.float32)
    ends_ref[...] = (starts + pc).astype(jnp.int32)

    rank1 = jnp.sum(a1 * c, axis=1, keepdims=True)
    rank2 = jnp.sum(a2 * c, axis=1, keepdims=True)
    st1 = jnp.sum(a1 * starts, axis=1, keepdims=True)
    st2 = jnp.sum(a2 * starts, axis=1, keepdims=True)
    pos1_ref[...] = (st1 + rank1).astype(jnp.int32)
    pos2_ref[...] = (st2 + rank2).astype(jnp.int32)


def _router(x, gate_w):
    return pl.pallas_call(
        _router_body,
        out_shape=(
            jax.ShapeDtypeStruct((T, 1), jnp.int32),
            jax.ShapeDtypeStruct((T, 1), jnp.int32),
            jax.ShapeDtypeStruct((T, 1), jnp.float32),
            jax.ShapeDtypeStruct((T, 1), jnp.float32),
            jax.ShapeDtypeStruct((1, LANES), jnp.int32),
        ),
    )(x, gate_w)


SCH = 16
NCH = TCH // SCH


def _dispatch_body(x_hbm, pos1_hbm, pos2_hbm, wn1_hbm, wn2_hbm, xs_hbm, ws_hbm,
                   p1_v, p2_v, p1f_v, p2f_v, w1f_v, w2f_v, ws_v,
                   rows, isems, o1sems, o2sems):
    cid = lax.axis_index("c")
    sid = lax.axis_index("s")
    wid = sid * NC + cid
    tb = wid * TCH

    pltpu.sync_copy(pos1_hbm.at[pl.ds(tb, TCH)], p1_v)
    pltpu.sync_copy(pos2_hbm.at[pl.ds(tb, TCH)], p2_v)

    icp = {}
    s1 = {}
    s2 = {}

    def start_in(ch):
        b = ch % 2
        icp[ch] = pltpu.async_copy(
            x_hbm.at[pl.ds(tb + ch * SCH, SCH)], rows[b], isems[b])

    start_in(0)
    for ch in range(NCH):
        if ch + 1 < NCH:
            if ch + 1 >= 2:
                s1[ch - 1].wait()
                s2[ch - 1].wait()
            start_in(ch + 1)
        b = ch % 2
        icp[ch].wait()
        idx1 = p1_v[pl.ds(ch * SCH, SCH)]
        idx2 = p2_v[pl.ds(ch * SCH, SCH)]
        s1[ch] = pltpu.async_copy(rows[b], xs_hbm.at[idx1], o1sems[b])
        s2[ch] = pltpu.async_copy(rows[b], xs_hbm.at[idx2], o2sems[b])
    for ch in (NCH - 2, NCH - 1):
        s1[ch].wait()
        s2[ch].wait()

    @pl.when((cid == 0) & (sid == 0))
    def _():
        pltpu.sync_copy(pos1_hbm, p1f_v)
        pltpu.sync_copy(pos2_hbm, p2f_v)
        pltpu.sync_copy(wn1_hbm, w1f_v)
        pltpu.sync_copy(wn2_hbm, w2f_v)

        def zinit(i, carry):
            ws_v[pl.ds(i * SCL, SCL)] = jnp.zeros((SCL,), jnp.float32)
            return carry

        lax.fori_loop(0, P // SCL, zinit, 0)

        def body(i, carry):
            q1 = p1f_v[pl.ds(i * SCL, SCL)]
            q2 = p2f_v[pl.ds(i * SCL, SCL)]
            plsc.store_scatter(ws_v, [q1], w1f_v[pl.ds(i * SCL, SCL)])
            plsc.store_scatter(ws_v, [q2], w2f_v[pl.ds(i * SCL, SCL)])
            return carry

        lax.fori_loop(0, T // SCL, body, 0)
        pltpu.sync_copy(ws_v, ws_hbm)


def _dispatch(x, pos1, pos2, wn1, wn2):
    return pl.kernel(
        _dispatch_body,
        out_type=(
            jax.ShapeDtypeStruct((P, D), jnp.float32),
            jax.ShapeDtypeStruct((P,), jnp.float32),
        ),
        mesh=plsc.VectorSubcoreMesh(core_axis_name="c", subcore_axis_name="s"),
        compiler_params=pltpu.CompilerParams(needs_layout_passes=False),
        scratch_types=[
            pltpu.VMEM((TCH,), jnp.int32),
            pltpu.VMEM((TCH,), jnp.int32),
            pltpu.VMEM((T,), jnp.int32),
            pltpu.VMEM((T,), jnp.int32),
            pltpu.VMEM((T,), jnp.float32),
            pltpu.VMEM((T,), jnp.float32),
            pltpu.VMEM((P,), jnp.float32),
            [pltpu.VMEM((SCH, D), jnp.float32) for _ in range(2)],
            [pltpu.SemaphoreType.DMA for _ in range(2)],
            [pltpu.SemaphoreType.DMA for _ in range(2)],
            [pltpu.SemaphoreType.DMA for _ in range(2)],
        ],
    )(x, pos1, pos2, wn1, wn2)


def _dot_bf16(a, b):
    return lax.dot_general(
        a, b, (((1,), (0,)), ((), ())),
        precision=lax.Precision.DEFAULT,
        preferred_element_type=jnp.float32)


def _mlp_body(bm_ref, xs_ref, ws_ref, w1_ref, w2_ref, ys_ref):
    @pl.when(pl.program_id(0) < bm_ref[NB])
    def _():
        h = _dot_bf16(xs_ref[...], w1_ref[0])
        act = h * jax.nn.sigmoid(h)
        act = act * ws_ref[...]
        ys_ref[...] = _dot_bf16(act, w2_ref[0])


def _mlp(bm, xs, ws, w1, w2):
    def _rowmap(b, bm):
        return (jnp.minimum(b, bm[NB] - 1), 0)

    grid_spec = pltpu.PrefetchScalarGridSpec(
        num_scalar_prefetch=1,
        grid=(NB,),
        in_specs=[
            pl.BlockSpec((BLK, D), _rowmap),
            pl.BlockSpec((BLK, 1), _rowmap),
            pl.BlockSpec((1, D, F), lambda b, bm: (bm[b], 0, 0)),
            pl.BlockSpec((1, F, D), lambda b, bm: (bm[b], 0, 0)),
        ],
        out_specs=pl.BlockSpec((BLK, D), _rowmap),
    )
    return pl.pallas_call(
        _mlp_body,
        grid_spec=grid_spec,
        out_shape=jax.ShapeDtypeStruct((P, D), jnp.float32),
        compiler_params=pltpu.CompilerParams(
            dimension_semantics=("arbitrary",),
        ),
    )(bm, xs, ws, w1, w2)


NCB = 6


def _combine_body(ys_hbm, pos1_hbm, pos2_hbm, out_hbm,
                  i1, i2, r1, r2, g1sems, g2sems, outsems):
    wid = lax.axis_index("s") * NC + lax.axis_index("c")
    base = wid * TCH
    nch = TCH // CCH
    pltpu.sync_copy(pos1_hbm.at[pl.ds(base, TCH)], i1)
    pltpu.sync_copy(pos2_hbm.at[pl.ds(base, TCH)], i2)

    g1 = {}
    g2 = {}
    oc = {}

    def fire(ch):
        b = ch % NCB
        g1[ch] = pltpu.async_copy(
            ys_hbm.at[i1.at[pl.ds(ch * CCH, CCH)]], r1[b], g1sems[b])
        g2[ch] = pltpu.async_copy(
            ys_hbm.at[i2.at[pl.ds(ch * CCH, CCH)]], r2[b], g2sems[b])

    for ch in range(min(NCB, nch)):
        fire(ch)
    for ch in range(nch):
        b = ch % NCB
        if ch >= 2:
            oc[ch - 2].wait()
            if ch + NCB - 2 < nch:
                fire(ch + NCB - 2)
        g1[ch].wait()
        g2[ch].wait()
        for r in range(CCH):
            def addvec(j, carry, r=r):
                for u in range(4):
                    sl = pl.ds((j * 4 + u) * SCL, SCL)
                    r1[b][r, sl] = r1[b][r, sl] + r2[b][r, sl]
                return carry
            lax.fori_loop(0, D // (SCL * 4), addvec, 0)
        oc[ch] = pltpu.async_copy(
            r1[b], out_hbm.at[pl.ds(base + ch * CCH, CCH)], outsems[ch % 2])
    for ch in (nch - 2, nch - 1):
        oc[ch].wait()


def _combine(ys, pos1, pos2):
    return pl.kernel(
        _combine_body,
        out_type=jax.ShapeDtypeStruct((T, D), jnp.float32),
        mesh=plsc.VectorSubcoreMesh(core_axis_name="c", subcore_axis_name="s"),
        compiler_params=pltpu.CompilerParams(needs_layout_passes=False),
        scratch_types=[
            pltpu.VMEM((TCH,), jnp.int32),
            pltpu.VMEM((TCH,), jnp.int32),
            [pltpu.VMEM((CCH, D), jnp.float32) for _ in range(NCB)],
            [pltpu.VMEM((CCH, D), jnp.float32) for _ in range(NCB)],
            [pltpu.SemaphoreType.DMA for _ in range(NCB)],
            [pltpu.SemaphoreType.DMA for _ in range(NCB)],
            [pltpu.SemaphoreType.DMA for _ in range(2)],
        ],
    )(ys, pos1, pos2)


def kernel(hidden_states, gate_w, experts_w1, experts_w2):
    pos1, pos2, wn1, wn2, ends = _router(hidden_states, gate_w)
    pos1f = pos1.reshape(T)
    pos2f = pos2.reshape(T)

    ends8 = ends[0, :E]
    bidx = jnp.arange(NB, dtype=jnp.int32) * BLK
    be = jnp.clip(jnp.sum((ends8[None, :] <= bidx[:, None]).astype(jnp.int32),
                          axis=1), 0, E - 1).astype(jnp.int32)
    nact = ends8[E - 1] // BLK
    be_sk = be[jnp.minimum(jnp.arange(NB), nact - 1)]
    bm = jnp.concatenate([be_sk, nact[None]]).astype(jnp.int32)

    xs, ws = _dispatch(hidden_states, pos1f, pos2f,
                       wn1.reshape(T), wn2.reshape(T))
    ys = _mlp(bm, xs, ws.reshape(P, 1), experts_w1, experts_w2)
    return _combine(ys, pos1f, pos2f)
